# baseline (device time: 8640 ns/iter reference)
import jax
import jax.numpy as jnp
from jax import lax
from jax.experimental import pallas as pl
from jax.experimental.pallas import tpu as pltpu

N_DEV = 8
N_CHUNKS = 4


def kernel(x):
    m_per, n = x.shape
    chunk = m_per // N_CHUNKS
    inv_total = 1.0 / (N_DEV * m_per)

    def body(x_ref, out_ref, acc_ref, buf_ref, copy_sems, send_sems, recv_sems):
        my_pos = lax.axis_index("i")

        def chunk_copy(k, slot):
            return pltpu.make_async_copy(
                x_ref.at[pl.ds(k * chunk, chunk), :],
                buf_ref.at[slot],
                copy_sems.at[slot],
            )

        chunk_copy(0, 0).start()
        chunk_copy(1, 1).start()

        barrier_sem = pltpu.get_barrier_semaphore()
        for j in range(N_DEV):
            @pl.when(my_pos != j)
            def _signal():
                pl.semaphore_signal(
                    barrier_sem, inc=1,
                    device_id=(j,), device_id_type=pl.DeviceIdType.MESH,
                )

        partial = None
        for k in range(N_CHUNKS):
            slot = k % 2
            chunk_copy(k, slot).wait()
            s = jnp.sum(buf_ref[slot], axis=0, keepdims=True)
            partial = s if partial is None else partial + s
            if k + 2 < N_CHUNKS:
                chunk_copy(k + 2, slot).start()
        acc_ref[pl.ds(my_pos, 1), :] = partial

        pl.semaphore_wait(barrier_sem, N_DEV - 1)

        for j in range(N_DEV):
            @pl.when(my_pos != j)
            def _send():
                rdma = pltpu.make_async_remote_copy(
                    src_ref=acc_ref.at[pl.ds(my_pos, 1), :],
                    dst_ref=acc_ref.at[pl.ds(my_pos, 1), :],
                    send_sem=send_sems.at[j],
                    recv_sem=recv_sems.at[my_pos],
                    device_id=(j,),
                    device_id_type=pl.DeviceIdType.MESH,
                )
                rdma.start()

        for j in range(N_DEV):
            @pl.when(my_pos != j)
            def _wait():
                rdma = pltpu.make_async_remote_copy(
                    src_ref=acc_ref.at[pl.ds(j, 1), :],
                    dst_ref=acc_ref.at[pl.ds(j, 1), :],
                    send_sem=send_sems.at[j],
                    recv_sem=recv_sems.at[j],
                    device_id=(j,),
                    device_id_type=pl.DeviceIdType.MESH,
                )
                rdma.wait_send()
                rdma.wait_recv()

        out_ref[:, :] = jnp.sum(acc_ref[:, :], axis=0, keepdims=True) * inv_total

    return pl.pallas_call(
        body,
        out_shape=jax.ShapeDtypeStruct((1, n), jnp.float32),
        in_specs=[pl.BlockSpec(memory_space=pl.ANY)],
        out_specs=pl.BlockSpec(memory_space=pltpu.VMEM),
        scratch_shapes=[
            pltpu.VMEM((N_DEV, n), jnp.float32),
            pltpu.VMEM((2, chunk, n), jnp.float32),
            pltpu.SemaphoreType.DMA((2,)),
            pltpu.SemaphoreType.DMA((N_DEV,)),
            pltpu.SemaphoreType.DMA((N_DEV,)),
        ],
        compiler_params=pltpu.CompilerParams(collective_id=0),
    )(x)


# device time: 8513 ns/iter; 1.0149x vs baseline; 1.0149x over previous
import jax
import jax.numpy as jnp
from jax import lax
from jax.experimental import pallas as pl
from jax.experimental.pallas import tpu as pltpu

N_DEV = 8


def kernel(x):
    m_per, n = x.shape
    inv_total = 1.0 / (N_DEV * m_per)

    def body(x_ref, out_ref, acc_ref, send_sems, recv_sems):
        my_pos = lax.axis_index("i")

        barrier_sem = pltpu.get_barrier_semaphore()
        for j in range(N_DEV):
            @pl.when(my_pos != j)
            def _signal():
                pl.semaphore_signal(
                    barrier_sem, inc=1,
                    device_id=(j,), device_id_type=pl.DeviceIdType.MESH,
                )

        acc_ref[pl.ds(my_pos, 1), :] = jnp.sum(x_ref[:, :], axis=0, keepdims=True)

        pl.semaphore_wait(barrier_sem, N_DEV - 1)

        for j in range(N_DEV):
            @pl.when(my_pos != j)
            def _send():
                rdma = pltpu.make_async_remote_copy(
                    src_ref=acc_ref.at[pl.ds(my_pos, 1), :],
                    dst_ref=acc_ref.at[pl.ds(my_pos, 1), :],
                    send_sem=send_sems.at[j],
                    recv_sem=recv_sems.at[my_pos],
                    device_id=(j,),
                    device_id_type=pl.DeviceIdType.MESH,
                )
                rdma.start()

        for j in range(N_DEV):
            @pl.when(my_pos != j)
            def _wait():
                rdma = pltpu.make_async_remote_copy(
                    src_ref=acc_ref.at[pl.ds(j, 1), :],
                    dst_ref=acc_ref.at[pl.ds(j, 1), :],
                    send_sem=send_sems.at[j],
                    recv_sem=recv_sems.at[j],
                    device_id=(j,),
                    device_id_type=pl.DeviceIdType.MESH,
                )
                rdma.wait_send()
                rdma.wait_recv()

        out_ref[:, :] = jnp.sum(acc_ref[:, :], axis=0, keepdims=True) * inv_total

    return pl.pallas_call(
        body,
        out_shape=jax.ShapeDtypeStruct((1, n), jnp.float32),
        in_specs=[pl.BlockSpec(memory_space=pltpu.VMEM)],
        out_specs=pl.BlockSpec(memory_space=pltpu.VMEM),
        scratch_shapes=[
            pltpu.VMEM((N_DEV, n), jnp.float32),
            pltpu.SemaphoreType.DMA((N_DEV,)),
            pltpu.SemaphoreType.DMA((N_DEV,)),
        ],
        compiler_params=pltpu.CompilerParams(collective_id=0),
    )(x)
